# baseline (speedup 1.0000x reference)
import functools

import jax
import jax.numpy as jnp
from jax import lax
from jax.experimental import pallas as pl
from jax.experimental.pallas import tpu as pltpu
from jax.experimental.pallas import tpu_sc as plsc

_NC = 2
_NS = 16
_NW = _NC * _NS
_CH = 128
_NBUF = 6
_BLK = 128


@functools.lru_cache(maxsize=None)
def _make_gather4(B, D, R):
    bpw = B // _NW
    mesh = plsc.VectorSubcoreMesh(core_axis_name="c", subcore_axis_name="s")

    @functools.partial(
        pl.kernel,
        mesh=mesh,
        out_type=tuple(
            jax.ShapeDtypeStruct((B, D), jnp.float32) for _ in range(4)
        ),
        scratch_types=[
            pltpu.VMEM((4, bpw), jnp.int32),
            pltpu.VMEM((_NBUF, _BLK, D), jnp.float32),
            pltpu.VMEM_SHARED((R, D), jnp.float32),
            pltpu.SemaphoreType.DMA,
            pltpu.SemaphoreType.DMA((_NBUF,)),
            pltpu.SemaphoreType.DMA((_NBUF,)),
        ],
    )
    def gather4(head_h, tail_h, rel_h, neg_h, ent_h, relt_h,
                out_h, out_r, out_t, out_n, idx_v, rows_v, relt_s,
                isem, gsem, ssem):
        cid = lax.axis_index("c")
        sid = lax.axis_index("s")
        wid = sid * _NC + cid
        base = wid * bpw
        tasks = (
            (head_h, ent_h, out_h, 0),
            (rel_h, relt_s, out_r, 1),
            (tail_h, ent_h, out_t, 2),
            (neg_h, ent_h, out_n, 3),
        )
        idx_copies = [
            pltpu.async_copy(idx_h.at[pl.ds(base, bpw)], idx_v.at[t], isem)
            for idx_h, _, _, t in tasks
        ]
        @pl.when(sid == 0)
        def _():
            pltpu.sync_copy(relt_h, relt_s)
        for cp in idx_copies:
            cp.wait()

        gpb = _BLK // _CH
        blk_per_task = bpw // _BLK
        order = (0, 2, 3, 1)
        blocks = [(t, v) for v in range(blk_per_task) for t in order]
        nblocks = len(blocks)
        first_rel = blocks.index((1, 0))

        def block_gathers(b, slot):
            t, v = blocks[b]
            table = tasks[t][1]
            return [
                pltpu.async_copy(
                    table.at[idx_v.at[t, pl.ds(v * _BLK + g * _CH, _CH)]],
                    rows_v.at[slot, pl.ds(g * _CH, _CH)],
                    gsem.at[slot],
                )
                for g in range(gpb)
            ]

        def block_store(b, slot):
            t, v = blocks[b]
            out = tasks[t][2]
            return pltpu.async_copy(
                rows_v.at[slot],
                out.at[pl.ds(base + v * _BLK, _BLK)],
                ssem.at[slot],
            )

        gathers = [None] * nblocks
        stores = [None] * nblocks
        for k in range(nblocks + _NBUF - 1):
            if k < nblocks:
                if k == first_rel:
                    plsc.subcore_barrier()
                slot = k % _NBUF
                if k >= _NBUF:
                    stores[k - _NBUF].wait()
                gathers[k] = block_gathers(k, slot)
            j = k - (_NBUF - 1)
            if 0 <= j < nblocks:
                slot = j % _NBUF
                for g in gathers[j]:
                    g.wait()
                stores[j] = block_store(j, slot)
        for j in range(max(0, nblocks - _NBUF), nblocks):
            stores[j].wait()

    return gather4


def kernel(head, tail, rel, neg, gpu_id, entity_emb, relation_emb):
    B = head.shape[0]
    D = entity_emb.shape[1]
    f = _make_gather4(B, D, relation_emb.shape[0])
    h, r, t, n = f(
        head.astype(jnp.int32),
        tail.astype(jnp.int32),
        rel.astype(jnp.int32),
        neg.astype(jnp.int32),
        entity_emb,
        relation_emb,
    )
    return (h, r, t, n)


# speedup vs baseline: 1.0050x; 1.0050x over previous
import functools

import jax
import jax.numpy as jnp
from jax import lax
from jax.experimental import pallas as pl
from jax.experimental.pallas import tpu as pltpu
from jax.experimental.pallas import tpu_sc as plsc

_NC = 2
_NS = 16
_NW = _NC * _NS
_CH = 128
_NBUF = 3
_BLK = 256


@functools.lru_cache(maxsize=None)
def _make_gather4(B, D, R):
    bpw = B // _NW
    mesh = plsc.VectorSubcoreMesh(core_axis_name="c", subcore_axis_name="s")

    @functools.partial(
        pl.kernel,
        mesh=mesh,
        out_type=tuple(
            jax.ShapeDtypeStruct((B, D), jnp.float32) for _ in range(4)
        ),
        scratch_types=[
            pltpu.VMEM((4, bpw), jnp.int32),
            pltpu.VMEM((_NBUF, _BLK, D), jnp.float32),
            pltpu.VMEM_SHARED((R, D), jnp.float32),
            pltpu.SemaphoreType.DMA,
            pltpu.SemaphoreType.DMA((_NBUF,)),
            pltpu.SemaphoreType.DMA((_NBUF,)),
        ],
    )
    def gather4(head_h, tail_h, rel_h, neg_h, ent_h, relt_h,
                out_h, out_r, out_t, out_n, idx_v, rows_v, relt_s,
                isem, gsem, ssem):
        cid = lax.axis_index("c")
        sid = lax.axis_index("s")
        wid = sid * _NC + cid
        base = wid * bpw
        tasks = (
            (head_h, ent_h, out_h, 0),
            (rel_h, relt_s, out_r, 1),
            (tail_h, ent_h, out_t, 2),
            (neg_h, ent_h, out_n, 3),
        )
        idx_copies = [
            pltpu.async_copy(idx_h.at[pl.ds(base, bpw)], idx_v.at[t], isem)
            for idx_h, _, _, t in tasks
        ]
        @pl.when(sid == 0)
        def _():
            pltpu.sync_copy(relt_h, relt_s)
        for cp in idx_copies:
            cp.wait()

        gpb = _BLK // _CH
        blk_per_task = bpw // _BLK
        order = (0, 2, 3, 1)
        blocks = [(t, v) for v in range(blk_per_task) for t in order]
        nblocks = len(blocks)
        first_rel = blocks.index((1, 0))

        def block_gathers(b, slot):
            t, v = blocks[b]
            table = tasks[t][1]
            return [
                pltpu.async_copy(
                    table.at[idx_v.at[t, pl.ds(v * _BLK + g * _CH, _CH)]],
                    rows_v.at[slot, pl.ds(g * _CH, _CH)],
                    gsem.at[slot],
                )
                for g in range(gpb)
            ]

        def block_store(b, slot):
            t, v = blocks[b]
            out = tasks[t][2]
            return pltpu.async_copy(
                rows_v.at[slot],
                out.at[pl.ds(base + v * _BLK, _BLK)],
                ssem.at[slot],
            )

        gathers = [None] * nblocks
        stores = [None] * nblocks
        for k in range(nblocks + _NBUF - 1):
            if k < nblocks:
                if k == first_rel:
                    plsc.subcore_barrier()
                slot = k % _NBUF
                if k >= _NBUF:
                    stores[k - _NBUF].wait()
                gathers[k] = block_gathers(k, slot)
            j = k - (_NBUF - 1)
            if 0 <= j < nblocks:
                slot = j % _NBUF
                for g in gathers[j]:
                    g.wait()
                stores[j] = block_store(j, slot)
        for j in range(max(0, nblocks - _NBUF), nblocks):
            stores[j].wait()

    return gather4


def kernel(head, tail, rel, neg, gpu_id, entity_emb, relation_emb):
    B = head.shape[0]
    D = entity_emb.shape[1]
    f = _make_gather4(B, D, relation_emb.shape[0])
    h, r, t, n = f(
        head.astype(jnp.int32),
        tail.astype(jnp.int32),
        rel.astype(jnp.int32),
        neg.astype(jnp.int32),
        entity_emb,
        relation_emb,
    )
    return (h, r, t, n)
